# baseline (speedup 1.0000x reference)
import functools

import jax
import jax.numpy as jnp
from jax import lax
from jax.experimental import pallas as pl
from jax.experimental.pallas import tpu as pltpu
from jax.experimental.pallas import tpu_sc as plsc

N_NODES_K = 10000
M_K = 5000
D_K = 128
N_EDGES_K = 320000
K_HOPS = 3

NW = 32
EPW = N_EDGES_K // NW
CHUNK = 80
NCHUNK = EPW // CHUNK
NG = CHUNK // 16
RPT = 624
TAIL = N_NODES_K - 16 * RPT
FCH = 48
NF = RPT // FCH


def _bcast_lane(vec, i):
    return lax.gather(
        vec, jnp.full((16, 1), i, dtype=jnp.int32),
        lax.GatherDimensionNumbers(
            offset_dims=(), collapsed_slice_dims=(0,), start_index_map=(0,)),
        slice_sizes=(1,),
        mode=lax.GatherScatterMode.PROMISE_IN_BOUNDS)


def _make_round(fold: bool):
    mesh = plsc.VectorSubcoreMesh(core_axis_name="c", subcore_axis_name="s")

    scratch = [
        pltpu.VMEM_SHARED((N_NODES_K, D_K), jnp.float32),
        pltpu.VMEM((4, 3, CHUNK), jnp.int32),
        pltpu.VMEM((4, CHUNK, D_K), jnp.float32),
        [pltpu.SemaphoreType.DMA] * 4,
        [pltpu.SemaphoreType.DMA] * 4,
        [pltpu.SemaphoreType.DMA] * 4,
    ]

    outs = [jax.ShapeDtypeStruct((N_NODES_K, D_K), jnp.float32),
            jax.ShapeDtypeStruct((N_NODES_K, D_K), jnp.float32)]
    if fold:
        outs.append(jax.ShapeDtypeStruct((2, N_NODES_K, D_K), jnp.float32))
    out_type = tuple(outs)

    def round_body(*refs):
        if fold:
            (p0h, p1h, comb_h, zeros_h, out0, out1, emb_h,
             acc, ring, rows, semI, semG, semS) = refs
        else:
            (t0h, comb_h, zeros_h, out0, out1,
             acc, ring, rows, semI, semG, semS) = refs
        c = lax.axis_index("c")
        s = lax.axis_index("s")
        wid = s * 2 + c
        tab = emb_h.at[c] if fold else t0h

        def load_idx(ci, r):
            pltpu.async_copy(comb_h.at[wid, ci], ring.at[r], semI[r])

        def wait_idx(r):
            pltpu.make_async_copy(comb_h.at[wid, 0], ring.at[r],
                                  semI[r]).wait()

        def issue_gather(r):
            pltpu.async_copy(tab.at[ring.at[r, 0]], rows.at[r], semG[r])

        def wait_gather(r):
            pltpu.make_async_copy(tab.at[ring.at[r, 0]], rows.at[r],
                                  semG[r]).wait()

        def scatter(r):
            pltpu.async_copy(rows.at[r], acc.at[ring.at[r, 1]], semS[r],
                             add=True)

        def wait_scatter(r):
            pltpu.make_async_copy(rows.at[r], acc.at[ring.at[r, 1]],
                                  semS[r]).wait()

        def scale(r):
            def grp_body(g, cc):
                e0 = g * 16
                wg = lax.bitcast_convert_type(
                    ring[r, 2, pl.ds(pl.multiple_of(e0, 16), 16)],
                    jnp.float32)
                for i in range(16):
                    e = e0 + i
                    wb = _bcast_lane(wg, i)
                    for j in range(8):
                        rows[r, e, pl.ds(j * 16, 16)] = (
                            rows[r, e, pl.ds(j * 16, 16)] * wb)
                return cc
            lax.fori_loop(0, NG, grp_body, 0)

        for m in range(4):
            load_idx(m, m)

        pltpu.sync_copy(zeros_h.at[pl.ds(s * RPT, RPT)],
                        acc.at[pl.ds(s * RPT, RPT)])

        @pl.when(s == 15)
        def _zero_tail():
            pltpu.sync_copy(zeros_h.at[pl.ds(16 * RPT, TAIL)],
                            acc.at[pl.ds(16 * RPT, TAIL)])

        if fold:
            fbase = s * RPT

            def fload(fi, a):
                r0 = fbase + fi * FCH
                pltpu.async_copy(p0h.at[pl.ds(r0, FCH)],
                                 rows.at[2 * a, pl.ds(0, FCH)], semG[2 * a])
                pltpu.async_copy(p1h.at[pl.ds(r0, FCH)],
                                 rows.at[2 * a + 1, pl.ds(0, FCH)],
                                 semG[2 * a + 1])

            def fwait(a):
                pltpu.make_async_copy(p0h.at[pl.ds(0, FCH)],
                                      rows.at[2 * a, pl.ds(0, FCH)],
                                      semG[2 * a]).wait()
                pltpu.make_async_copy(p1h.at[pl.ds(0, FCH)],
                                      rows.at[2 * a + 1, pl.ds(0, FCH)],
                                      semG[2 * a + 1]).wait()

            def fcomp(a):
                def frow(rr, cc):
                    for j in range(8):
                        rows[2 * a, rr, pl.ds(j * 16, 16)] = (
                            rows[2 * a, rr, pl.ds(j * 16, 16)]
                            + rows[2 * a + 1, rr, pl.ds(j * 16, 16)])
                    return cc
                lax.fori_loop(0, FCH, frow, 0)

            def fstore(fi, a):
                r0 = fbase + fi * FCH
                pltpu.async_copy(rows.at[2 * a, pl.ds(0, FCH)],
                                 emb_h.at[c, pl.ds(r0, FCH)], semS[a])

            def fwait_store(a):
                pltpu.make_async_copy(rows.at[2 * a, pl.ds(0, FCH)],
                                      emb_h.at[c, pl.ds(0, FCH)],
                                      semS[a]).wait()

            fload(0, 0)

            def fpair(q, cc):
                fi0 = q * 2

                @pl.when(q > 0)
                def _ws1():
                    fwait_store(1)
                fload(fi0 + 1, 1)
                fwait(0)
                fcomp(0)
                fstore(fi0, 0)
                fwait_store(0)

                @pl.when(fi0 + 2 < NF)
                def _next():
                    fload(fi0 + 2, 0)
                fwait(1)
                fcomp(1)
                fstore(fi0 + 1, 1)
                return cc

            lax.fori_loop(0, NF // 2, fpair, 0)
            fwait_store(1)
            fwait(0)
            fcomp(0)
            fstore(NF - 1, 0)
            fwait_store(0)

            @pl.when(s == 15)
            def _fold_tail():
                pltpu.sync_copy(p0h.at[pl.ds(16 * RPT, TAIL)],
                                rows.at[0, pl.ds(0, TAIL)])
                pltpu.sync_copy(p1h.at[pl.ds(16 * RPT, TAIL)],
                                rows.at[1, pl.ds(0, TAIL)])

                def trow(rr, cc):
                    for j in range(8):
                        rows[0, rr, pl.ds(j * 16, 16)] = (
                            rows[0, rr, pl.ds(j * 16, 16)]
                            + rows[1, rr, pl.ds(j * 16, 16)])
                    return cc
                lax.fori_loop(0, TAIL, trow, 0)
                pltpu.sync_copy(rows.at[0, pl.ds(0, TAIL)],
                                emb_h.at[c, pl.ds(16 * RPT, TAIL)])

        plsc.subcore_barrier()

        wait_idx(0)
        wait_idx(1)
        issue_gather(0)
        issue_gather(1)

        def process(k, p, first=False):
            @pl.when(k + 2 < NCHUNK)
            def _pref_gather():
                wait_idx((p + 2) % 4)
                issue_gather((p + 2) % 4)
            wait_gather(p)
            scale(p)
            scatter(p)
            if not first:
                wait_scatter((p - 1) % 4)

            @pl.when(k + 3 < NCHUNK)
            def _pref_idx():
                if not (first and p == 0):
                    load_idx(k + 3, (p + 3) % 4)

        process(0, 0, first=True)
        for i in range(1, 4):
            process(i, i)

        def body(b, cc):
            k0 = b * 4
            for i in range(4):
                process(k0 + i, i)
            return cc

        lax.fori_loop(1, NCHUNK // 4, body, 0)
        process(NCHUNK - 1, 0)
        wait_scatter(0)

        plsc.subcore_barrier()

        @pl.when(c == 0)
        def _dump0():
            pltpu.sync_copy(acc.at[pl.ds(s * RPT, RPT)],
                            out0.at[pl.ds(s * RPT, RPT)])

            @pl.when(s == 15)
            def _tail0():
                pltpu.sync_copy(acc.at[pl.ds(16 * RPT, TAIL)],
                                out0.at[pl.ds(16 * RPT, TAIL)])

        @pl.when(c == 1)
        def _dump1():
            pltpu.sync_copy(acc.at[pl.ds(s * RPT, RPT)],
                            out1.at[pl.ds(s * RPT, RPT)])

            @pl.when(s == 15)
            def _tail1():
                pltpu.sync_copy(acc.at[pl.ds(16 * RPT, TAIL)],
                                out1.at[pl.ds(16 * RPT, TAIL)])

    return functools.partial(
        pl.kernel, mesh=mesh, out_type=out_type, scratch_types=scratch,
    )(round_body)


_round_one = _make_round(fold=False)
_round_two = _make_round(fold=True)


def _mm_body(u0_ref, u1_ref, i0_ref, i1_ref, o_ref):
    a = u0_ref[...] + u1_ref[...]
    b = i0_ref[...] + i1_ref[...]
    o_ref[...] = lax.dot_general(a, b, (((1,), (1,)), ((), ())),
                                 preferred_element_type=jnp.float32)


def _rating(p0, p1):
    bm = 1000
    g = M_K // bm

    return pl.pallas_call(
        _mm_body,
        grid=(g,),
        in_specs=[
            pl.BlockSpec((bm, D_K), lambda i: (i, 0)),
            pl.BlockSpec((bm, D_K), lambda i: (i, 0)),
            pl.BlockSpec((M_K, D_K), lambda i: (1, 0)),
            pl.BlockSpec((M_K, D_K), lambda i: (1, 0)),
        ],
        out_specs=pl.BlockSpec((bm, M_K), lambda i: (i, 0)),
        out_shape=jax.ShapeDtypeStruct((M_K, M_K), jnp.float32),
    )(p0, p1, p0, p1)


def kernel(E0, edge_weight, edge_index):
    src = edge_index[0].reshape(NW, NCHUNK, CHUNK)
    dst = edge_index[1].reshape(NW, NCHUNK, CHUNK)
    wbits = lax.bitcast_convert_type(
        edge_weight.reshape(NW, NCHUNK, CHUNK), jnp.int32)
    comb = jnp.stack([src, dst, wbits], axis=2)
    zeros = jnp.zeros((N_NODES_K, D_K), jnp.float32)
    p0, p1 = _round_one(E0, comb, zeros)
    for _ in range(K_HOPS - 1):
        p0, p1, _unused = _round_two(p0, p1, comb, zeros)
    return _rating(p0, p1)


# speedup vs baseline: 1.0408x; 1.0408x over previous
import functools

import jax
import jax.numpy as jnp
from jax import lax
from jax.experimental import pallas as pl
from jax.experimental.pallas import tpu as pltpu
from jax.experimental.pallas import tpu_sc as plsc

N_NODES_K = 10000
M_K = 5000
D_K = 128
N_EDGES_K = 320000
K_HOPS = 3

NW = 32
EPW = N_EDGES_K // NW
CHUNK = 80
NCHUNK = EPW // CHUNK
NG = CHUNK // 16
RPT = 624
TAIL = N_NODES_K - 16 * RPT
FCH = 48
NF = RPT // FCH


def _bcast_lane(vec, i):
    return lax.gather(
        vec, jnp.full((16, 1), i, dtype=jnp.int32),
        lax.GatherDimensionNumbers(
            offset_dims=(), collapsed_slice_dims=(0,), start_index_map=(0,)),
        slice_sizes=(1,),
        mode=lax.GatherScatterMode.PROMISE_IN_BOUNDS)


def _make_round(fold: bool):
    mesh = plsc.VectorSubcoreMesh(core_axis_name="c", subcore_axis_name="s")

    scratch = [
        pltpu.VMEM_SHARED((N_NODES_K, D_K), jnp.float32),
        pltpu.VMEM((4, CHUNK), jnp.int32),
        pltpu.VMEM((4, CHUNK), jnp.int32),
        pltpu.VMEM((4, CHUNK), jnp.float32),
        pltpu.VMEM((4, CHUNK, D_K), jnp.float32),
        [pltpu.SemaphoreType.DMA] * 4,
        [pltpu.SemaphoreType.DMA] * 4,
        [pltpu.SemaphoreType.DMA] * 4,
    ]

    outs = [jax.ShapeDtypeStruct((N_NODES_K, D_K), jnp.float32),
            jax.ShapeDtypeStruct((N_NODES_K, D_K), jnp.float32)]
    if fold:
        outs.append(jax.ShapeDtypeStruct((2, N_NODES_K, D_K), jnp.float32))
    out_type = tuple(outs)

    def round_body(*refs):
        if fold:
            (p0h, p1h, src_h, dst_h, w_h, zeros_h, out0, out1, emb_h,
             acc, src_v, dst_v, w_v, rows, semI, semG, semS) = refs
        else:
            (t0h, src_h, dst_h, w_h, zeros_h, out0, out1,
             acc, src_v, dst_v, w_v, rows, semI, semG, semS) = refs
        c = lax.axis_index("c")
        s = lax.axis_index("s")
        wid = s * 2 + c
        tab = emb_h.at[c] if fold else t0h

        def load_idx(ci, r):
            pltpu.async_copy(src_h.at[wid, ci], src_v.at[r], semI[r])
            pltpu.async_copy(dst_h.at[wid, ci], dst_v.at[r], semI[r])
            pltpu.async_copy(w_h.at[wid, ci], w_v.at[r], semI[r])

        def wait_idx(r):
            pltpu.make_async_copy(src_h.at[wid, 0], src_v.at[r],
                                  semI[r]).wait()
            pltpu.make_async_copy(dst_h.at[wid, 0], dst_v.at[r],
                                  semI[r]).wait()
            pltpu.make_async_copy(w_h.at[wid, 0], w_v.at[r], semI[r]).wait()

        def issue_gather(r):
            pltpu.async_copy(tab.at[src_v.at[r]], rows.at[r], semG[r])

        def wait_gather(r):
            pltpu.make_async_copy(tab.at[src_v.at[r]], rows.at[r],
                                  semG[r]).wait()

        def scatter(r):
            pltpu.async_copy(rows.at[r], acc.at[dst_v.at[r]], semS[r],
                             add=True)

        def wait_scatter(r):
            pltpu.make_async_copy(rows.at[r], acc.at[dst_v.at[r]],
                                  semS[r]).wait()

        def scale(r):
            def grp_body(g, cc):
                e0 = g * 16
                wg = w_v[r, pl.ds(pl.multiple_of(e0, 16), 16)]
                for i in range(16):
                    e = e0 + i
                    wb = _bcast_lane(wg, i)
                    for j in range(8):
                        rows[r, e, pl.ds(j * 16, 16)] = (
                            rows[r, e, pl.ds(j * 16, 16)] * wb)
                return cc
            lax.fori_loop(0, NG, grp_body, 0)

        for m in range(4):
            load_idx(m, m)

        pltpu.sync_copy(zeros_h.at[pl.ds(s * RPT, RPT)],
                        acc.at[pl.ds(s * RPT, RPT)])

        @pl.when(s == 15)
        def _zero_tail():
            pltpu.sync_copy(zeros_h.at[pl.ds(16 * RPT, TAIL)],
                            acc.at[pl.ds(16 * RPT, TAIL)])

        if fold:
            fbase = s * RPT

            def fload(fi, a):
                r0 = fbase + fi * FCH
                pltpu.async_copy(p0h.at[pl.ds(r0, FCH)],
                                 rows.at[2 * a, pl.ds(0, FCH)], semG[2 * a])
                pltpu.async_copy(p1h.at[pl.ds(r0, FCH)],
                                 rows.at[2 * a + 1, pl.ds(0, FCH)],
                                 semG[2 * a + 1])

            def fwait(a):
                pltpu.make_async_copy(p0h.at[pl.ds(0, FCH)],
                                      rows.at[2 * a, pl.ds(0, FCH)],
                                      semG[2 * a]).wait()
                pltpu.make_async_copy(p1h.at[pl.ds(0, FCH)],
                                      rows.at[2 * a + 1, pl.ds(0, FCH)],
                                      semG[2 * a + 1]).wait()

            def fcomp(a):
                def frow(rr, cc):
                    for j in range(8):
                        rows[2 * a, rr, pl.ds(j * 16, 16)] = (
                            rows[2 * a, rr, pl.ds(j * 16, 16)]
                            + rows[2 * a + 1, rr, pl.ds(j * 16, 16)])
                    return cc
                lax.fori_loop(0, FCH, frow, 0)

            def fstore(fi, a):
                r0 = fbase + fi * FCH
                pltpu.async_copy(rows.at[2 * a, pl.ds(0, FCH)],
                                 emb_h.at[c, pl.ds(r0, FCH)], semS[a])

            def fwait_store(a):
                pltpu.make_async_copy(rows.at[2 * a, pl.ds(0, FCH)],
                                      emb_h.at[c, pl.ds(0, FCH)],
                                      semS[a]).wait()

            fload(0, 0)

            def fpair(q, cc):
                fi0 = q * 2

                @pl.when(q > 0)
                def _ws1():
                    fwait_store(1)
                fload(fi0 + 1, 1)
                fwait(0)
                fcomp(0)
                fstore(fi0, 0)
                fwait_store(0)

                @pl.when(fi0 + 2 < NF)
                def _next():
                    fload(fi0 + 2, 0)
                fwait(1)
                fcomp(1)
                fstore(fi0 + 1, 1)
                return cc

            lax.fori_loop(0, NF // 2, fpair, 0)
            fwait_store(1)
            fwait(0)
            fcomp(0)
            fstore(NF - 1, 0)
            fwait_store(0)

            @pl.when(s == 15)
            def _fold_tail():
                pltpu.sync_copy(p0h.at[pl.ds(16 * RPT, TAIL)],
                                rows.at[0, pl.ds(0, TAIL)])
                pltpu.sync_copy(p1h.at[pl.ds(16 * RPT, TAIL)],
                                rows.at[1, pl.ds(0, TAIL)])

                def trow(rr, cc):
                    for j in range(8):
                        rows[0, rr, pl.ds(j * 16, 16)] = (
                            rows[0, rr, pl.ds(j * 16, 16)]
                            + rows[1, rr, pl.ds(j * 16, 16)])
                    return cc
                lax.fori_loop(0, TAIL, trow, 0)
                pltpu.sync_copy(rows.at[0, pl.ds(0, TAIL)],
                                emb_h.at[c, pl.ds(16 * RPT, TAIL)])

        plsc.subcore_barrier()

        wait_idx(0)
        wait_idx(1)
        issue_gather(0)
        issue_gather(1)

        def process(k, p, first=False):
            @pl.when(k + 2 < NCHUNK)
            def _pref_gather():
                wait_idx((p + 2) % 4)
                issue_gather((p + 2) % 4)
            wait_gather(p)
            scale(p)
            scatter(p)
            if not first:
                wait_scatter((p - 1) % 4)

            @pl.when(k + 3 < NCHUNK)
            def _pref_idx():
                if not (first and p == 0):
                    load_idx(k + 3, (p + 3) % 4)

        process(0, 0, first=True)
        for i in range(1, 4):
            process(i, i)

        def body(b, cc):
            k0 = b * 4
            for i in range(4):
                process(k0 + i, i)
            return cc

        lax.fori_loop(1, NCHUNK // 4, body, 0)
        process(NCHUNK - 1, 0)
        wait_scatter(0)

        plsc.subcore_barrier()

        @pl.when(c == 0)
        def _dump0():
            pltpu.sync_copy(acc.at[pl.ds(s * RPT, RPT)],
                            out0.at[pl.ds(s * RPT, RPT)])

            @pl.when(s == 15)
            def _tail0():
                pltpu.sync_copy(acc.at[pl.ds(16 * RPT, TAIL)],
                                out0.at[pl.ds(16 * RPT, TAIL)])

        @pl.when(c == 1)
        def _dump1():
            pltpu.sync_copy(acc.at[pl.ds(s * RPT, RPT)],
                            out1.at[pl.ds(s * RPT, RPT)])

            @pl.when(s == 15)
            def _tail1():
                pltpu.sync_copy(acc.at[pl.ds(16 * RPT, TAIL)],
                                out1.at[pl.ds(16 * RPT, TAIL)])

    return functools.partial(
        pl.kernel, mesh=mesh, out_type=out_type, scratch_types=scratch,
    )(round_body)


_round_one = _make_round(fold=False)
_round_two = _make_round(fold=True)


def _mm_body(u0_ref, u1_ref, i0_ref, i1_ref, o_ref):
    a = u0_ref[...] + u1_ref[...]
    b = i0_ref[...] + i1_ref[...]
    o_ref[...] = lax.dot_general(a, b, (((1,), (1,)), ((), ())),
                                 preferred_element_type=jnp.float32)


def _rating(p0, p1):
    bm = 1000
    g = M_K // bm

    return pl.pallas_call(
        _mm_body,
        grid=(g,),
        in_specs=[
            pl.BlockSpec((bm, D_K), lambda i: (i, 0)),
            pl.BlockSpec((bm, D_K), lambda i: (i, 0)),
            pl.BlockSpec((M_K, D_K), lambda i: (1, 0)),
            pl.BlockSpec((M_K, D_K), lambda i: (1, 0)),
        ],
        out_specs=pl.BlockSpec((bm, M_K), lambda i: (i, 0)),
        out_shape=jax.ShapeDtypeStruct((M_K, M_K), jnp.float32),
    )(p0, p1, p0, p1)


def kernel(E0, edge_weight, edge_index):
    src = edge_index[0].reshape(NW, NCHUNK, CHUNK)
    dst = edge_index[1].reshape(NW, NCHUNK, CHUNK)
    w = edge_weight.reshape(NW, NCHUNK, CHUNK)
    zeros = jnp.zeros((N_NODES_K, D_K), jnp.float32)
    p0, p1 = _round_one(E0, src, dst, w, zeros)
    for _ in range(K_HOPS - 1):
        p0, p1, _unused = _round_two(p0, p1, src, dst, w, zeros)
    return _rating(p0, p1)
